# Initial kernel scaffold; baseline (speedup 1.0000x reference)
import jax
import jax.numpy as jnp
from jax.experimental import pallas as pl


def kernel(inputs, Wg, W1, b1, W2, b2):
    raise NotImplementedError("write your pallas kernel here")


# speedup vs baseline: 1.7447x; 1.7447x over previous
import functools

import jax
import jax.numpy as jnp
from jax.experimental import pallas as pl
from jax.experimental.pallas import tpu as pltpu

E = 8
K = 2


def _gate_body(x_ref, wg_ref, comb_ref):
    x = x_ref[...]
    logits = jnp.dot(x, wg_ref[...], preferred_element_type=jnp.float32)
    eidx = jax.lax.broadcasted_iota(jnp.int32, logits.shape, 1)
    m1 = jnp.max(logits, axis=1, keepdims=True)
    i1 = jnp.min(jnp.where(logits == m1, eidx, E), axis=1, keepdims=True)
    masked = jnp.where(eidx == i1, -jnp.inf, logits)
    m2 = jnp.max(masked, axis=1, keepdims=True)
    i2 = jnp.min(jnp.where(masked == m2, eidx, E), axis=1, keepdims=True)
    e2 = jnp.exp(m2 - m1)
    w1 = 1.0 / (1.0 + e2)
    w2 = e2 / (1.0 + e2)
    comb_ref[...] = jnp.where(eidx == i1, w1, jnp.where(eidx == i2, w2, 0.0))


def _ffn_dense_body(comb_ref, x_ref, w1_ref, b1_ref, w2_ref, b2_ref, out_ref,
                    *, t_chunk):
    e = pl.program_id(0)

    @pl.when(e == 0)
    def _():
        out_ref[...] = jnp.zeros_like(out_ref)

    T = x_ref.shape[0]
    b1 = b1_ref[0, 0, :]
    b2 = b2_ref[0, 0, :]
    for i in range(T // t_chunk):
        sl = pl.ds(i * t_chunk, t_chunk)
        xs = x_ref[sl, :]
        h = jnp.dot(xs, w1_ref[0], preferred_element_type=jnp.float32) + b1[None, :]
        h = jax.nn.gelu(h)
        o = jnp.dot(h, w2_ref[0], preferred_element_type=jnp.float32) + b2[None, :]
        c = comb_ref[0, 0, sl]
        out_ref[sl, :] += c[:, None] * o


def kernel(inputs, Wg, W1, b1, W2, b2):
    B, S, D = inputs.shape
    T = B * S
    F = W1.shape[2]
    x = inputs.reshape(T, D)

    comb = pl.pallas_call(
        _gate_body,
        out_shape=jax.ShapeDtypeStruct((T, E), jnp.float32),
    )(x, Wg)

    comb_t = comb.T.reshape(E, 1, T)

    out = pl.pallas_call(
        functools.partial(_ffn_dense_body, t_chunk=512),
        grid=(E,),
        in_specs=[
            pl.BlockSpec((1, 1, T), lambda e: (e, 0, 0)),
            pl.BlockSpec((T, D), lambda e: (0, 0)),
            pl.BlockSpec((1, D, F), lambda e: (e, 0, 0)),
            pl.BlockSpec((1, 1, F), lambda e: (e, 0, 0)),
            pl.BlockSpec((1, F, D), lambda e: (e, 0, 0)),
            pl.BlockSpec((1, 1, D), lambda e: (e, 0, 0)),
        ],
        out_specs=pl.BlockSpec((T, D), lambda e: (0, 0)),
        out_shape=jax.ShapeDtypeStruct((T, D), jnp.float32),
    )(comb_t, x, W1, b1.reshape(E, 1, F), W2, b2.reshape(E, 1, D))

    return out.reshape(B, S, D)


# speedup vs baseline: 1.7704x; 1.0147x over previous
import functools

import jax
import jax.numpy as jnp
from jax import lax
from jax.experimental import pallas as pl
from jax.experimental.pallas import tpu as pltpu
from jax.experimental.pallas import tpu_sc as plsc

E = 8
K = 2
BLK = 128
NC = 2
NS = 16
NW = NC * NS


def _gate_body(x_ref, wg_ref, p0_ref, p1_ref, w0_ref, w1_ref, cnt_ref):
    T = x_ref.shape[0]
    x = x_ref[...]
    logits = jnp.dot(x, wg_ref[...], preferred_element_type=jnp.float32)
    eidx = lax.broadcasted_iota(jnp.int32, logits.shape, 1)
    m1 = jnp.max(logits, axis=1, keepdims=True)
    i1 = jnp.min(jnp.where(logits == m1, eidx, E), axis=1, keepdims=True)
    masked = jnp.where(eidx == i1, -jnp.inf, logits)
    m2 = jnp.max(masked, axis=1, keepdims=True)
    i2 = jnp.min(jnp.where(masked == m2, eidx, E), axis=1, keepdims=True)
    e2 = jnp.exp(m2 - m1)
    w0_ref[...] = 1.0 / (1.0 + e2)
    w1_ref[...] = e2 / (1.0 + e2)

    mask1 = (eidx == i1).astype(jnp.float32)
    mask2 = (eidx == i2).astype(jnp.float32)
    m = mask1 + mask2
    rows = lax.broadcasted_iota(jnp.int32, (T, T), 0)
    cols = lax.broadcasted_iota(jnp.int32, (T, T), 1)
    tri = (rows > cols).astype(jnp.float32)
    excl = jnp.dot(tri, m, preferred_element_type=jnp.float32)
    counts = jnp.sum(m, axis=0, keepdims=True)
    nblk = jnp.floor((counts + (BLK - 1)) * (1.0 / BLK))
    r8 = lax.broadcasted_iota(jnp.int32, (E, E), 0)
    c8 = lax.broadcasted_iota(jnp.int32, (E, E), 1)
    tri8 = (r8 < c8).astype(jnp.float32)
    seg = jnp.dot(nblk, tri8, preferred_element_type=jnp.float32) * BLK
    dest = seg + excl
    p0_ref[...] = jnp.sum(mask1 * dest, axis=1, keepdims=True).astype(jnp.int32)
    p1_ref[...] = jnp.sum(mask2 * dest, axis=1, keepdims=True).astype(jnp.int32)
    cnt_ref[...] = counts.astype(jnp.int32)


def _dispatch_body(x_hbm, p0_hbm, p1_hbm, xs_hbm, pos_v, xbuf_v, sem):
    wid = lax.axis_index("s") * NC + lax.axis_index("c")
    base = wid * 64
    pltpu.sync_copy(x_hbm.at[pl.ds(base, 64)], xbuf_v)
    pltpu.sync_copy(p0_hbm.at[pl.ds(base, 64)], pos_v)
    pltpu.async_copy(xbuf_v, xs_hbm.at[pos_v], sem).wait()
    pltpu.sync_copy(p1_hbm.at[pl.ds(base, 64)], pos_v)
    pltpu.async_copy(xbuf_v, xs_hbm.at[pos_v], sem).wait()


def _ffn_group_body(gid_ref, xs_ref, w1_ref, b1_ref, w2_ref, b2_ref, y_ref):
    h = jnp.dot(xs_ref[...], w1_ref[0], preferred_element_type=jnp.float32)
    h = jax.nn.gelu(h + b1_ref[0, 0, :][None, :])
    y = jnp.dot(h, w2_ref[0], preferred_element_type=jnp.float32)
    y_ref[...] = y + b2_ref[0, 0, :][None, :]


def _gather_back_body(p0_hbm, p1_hbm, ys_hbm, y0_hbm, y1_hbm, pos_v, ybuf_v, sem):
    wid = lax.axis_index("s") * NC + lax.axis_index("c")
    base = wid * 64
    pltpu.sync_copy(p0_hbm.at[pl.ds(base, 64)], pos_v)
    pltpu.async_copy(ys_hbm.at[pos_v], ybuf_v, sem).wait()
    pltpu.sync_copy(ybuf_v, y0_hbm.at[pl.ds(base, 64)])
    pltpu.sync_copy(p1_hbm.at[pl.ds(base, 64)], pos_v)
    pltpu.async_copy(ys_hbm.at[pos_v], ybuf_v, sem).wait()
    pltpu.sync_copy(ybuf_v, y1_hbm.at[pl.ds(base, 64)])


def _combine_body(w0_ref, w1_ref, y0_ref, y1_ref, out_ref):
    out_ref[...] = w0_ref[...] * y0_ref[...] + w1_ref[...] * y1_ref[...]


def kernel(inputs, Wg, W1, b1, W2, b2):
    B, S, D = inputs.shape
    T = B * S
    F = W1.shape[2]
    x = inputs.reshape(T, D)
    NB = T * K // BLK + E
    PMAX = NB * BLK

    p0c, p1c, w0c, w1c, counts = pl.pallas_call(
        _gate_body,
        out_shape=(
            jax.ShapeDtypeStruct((T, 1), jnp.int32),
            jax.ShapeDtypeStruct((T, 1), jnp.int32),
            jax.ShapeDtypeStruct((T, 1), jnp.float32),
            jax.ShapeDtypeStruct((T, 1), jnp.float32),
            jax.ShapeDtypeStruct((1, E), jnp.int32),
        ),
    )(x, Wg)
    p0 = p0c.reshape(T)
    p1 = p1c.reshape(T)

    cnt = counts.reshape(E)
    nblk = (cnt + BLK - 1) // BLK
    ends = jnp.cumsum(nblk)
    bid = jnp.arange(NB, dtype=jnp.int32)
    gid = jnp.sum((bid[:, None] >= ends[None, :]).astype(jnp.int32), axis=1)
    last_ne = jnp.max(jnp.where(cnt > 0, jnp.arange(E, dtype=jnp.int32), 0))
    gid = jnp.minimum(gid, last_ne)

    mesh = plsc.VectorSubcoreMesh(core_axis_name="c", subcore_axis_name="s")
    x_sorted = pl.kernel(
        _dispatch_body,
        mesh=mesh,
        out_type=jax.ShapeDtypeStruct((PMAX, D), jnp.float32),
        scratch_types=[
            pltpu.VMEM((64,), jnp.int32),
            pltpu.VMEM((64, D), jnp.float32),
            pltpu.SemaphoreType.DMA,
        ],
    )(x, p0, p1)

    y_sorted = pl.pallas_call(
        _ffn_group_body,
        grid_spec=pltpu.PrefetchScalarGridSpec(
            num_scalar_prefetch=1,
            grid=(NB,),
            in_specs=[
                pl.BlockSpec((BLK, D), lambda i, g: (i, 0)),
                pl.BlockSpec((1, D, F), lambda i, g: (g[i], 0, 0)),
                pl.BlockSpec((1, 1, F), lambda i, g: (g[i], 0, 0)),
                pl.BlockSpec((1, F, D), lambda i, g: (g[i], 0, 0)),
                pl.BlockSpec((1, 1, D), lambda i, g: (g[i], 0, 0)),
            ],
            out_specs=pl.BlockSpec((BLK, D), lambda i, g: (i, 0)),
        ),
        out_shape=jax.ShapeDtypeStruct((PMAX, D), jnp.float32),
    )(gid, x_sorted, W1, b1.reshape(E, 1, F), W2, b2.reshape(E, 1, D))

    y0, y1 = pl.kernel(
        _gather_back_body,
        mesh=mesh,
        out_type=(
            jax.ShapeDtypeStruct((T, D), jnp.float32),
            jax.ShapeDtypeStruct((T, D), jnp.float32),
        ),
        scratch_types=[
            pltpu.VMEM((64,), jnp.int32),
            pltpu.VMEM((64, D), jnp.float32),
            pltpu.SemaphoreType.DMA,
        ],
    )(p0, p1, y_sorted)

    TB = 512
    out = pl.pallas_call(
        _combine_body,
        grid=(T // TB,),
        in_specs=[
            pl.BlockSpec((TB, 1), lambda i: (i, 0)),
            pl.BlockSpec((TB, 1), lambda i: (i, 0)),
            pl.BlockSpec((TB, D), lambda i: (i, 0)),
            pl.BlockSpec((TB, D), lambda i: (i, 0)),
        ],
        out_specs=pl.BlockSpec((TB, D), lambda i: (i, 0)),
        out_shape=jax.ShapeDtypeStruct((T, D), jnp.float32),
    )(w0c, w1c, y0, y1)

    return out.reshape(B, S, D)


# speedup vs baseline: 1.8172x; 1.0264x over previous
import functools

import jax
import jax.numpy as jnp
from jax import lax
from jax.experimental import pallas as pl
from jax.experimental.pallas import tpu as pltpu
from jax.experimental.pallas import tpu_sc as plsc

E = 8
K = 2
BLK = 256
NC = 2
NS = 16
NW = NC * NS


def _gate_body(x_ref, wg_ref, p0_ref, p1_ref, w0_ref, w1_ref, cnt_ref):
    T = x_ref.shape[0]
    x = x_ref[...]
    logits = jnp.dot(x, wg_ref[...], preferred_element_type=jnp.float32)
    eidx = lax.broadcasted_iota(jnp.int32, logits.shape, 1)
    m1 = jnp.max(logits, axis=1, keepdims=True)
    i1 = jnp.min(jnp.where(logits == m1, eidx, E), axis=1, keepdims=True)
    masked = jnp.where(eidx == i1, -jnp.inf, logits)
    m2 = jnp.max(masked, axis=1, keepdims=True)
    i2 = jnp.min(jnp.where(masked == m2, eidx, E), axis=1, keepdims=True)
    e2 = jnp.exp(m2 - m1)
    w0_ref[...] = 1.0 / (1.0 + e2)
    w1_ref[...] = e2 / (1.0 + e2)

    mask1 = (eidx == i1).astype(jnp.float32)
    mask2 = (eidx == i2).astype(jnp.float32)
    m = mask1 + mask2
    C = 256
    rows = lax.broadcasted_iota(jnp.int32, (C, C), 0)
    cols = lax.broadcasted_iota(jnp.int32, (C, C), 1)
    tri = (rows > cols).astype(jnp.float32)
    carry = jnp.zeros((1, E), jnp.float32)
    chunks = []
    for c in range(T // C):
        mc = m[c * C:(c + 1) * C, :]
        chunks.append(jnp.dot(tri, mc, preferred_element_type=jnp.float32)
                      + carry)
        carry = carry + jnp.sum(mc, axis=0, keepdims=True)
    excl = jnp.concatenate(chunks, axis=0)
    counts = carry
    nblk = jnp.floor((counts + (BLK - 1)) * (1.0 / BLK))
    r8 = lax.broadcasted_iota(jnp.int32, (E, E), 0)
    c8 = lax.broadcasted_iota(jnp.int32, (E, E), 1)
    tri8 = (r8 < c8).astype(jnp.float32)
    seg = jnp.dot(nblk, tri8, preferred_element_type=jnp.float32) * BLK
    dest = seg + excl
    p0_ref[...] = jnp.sum(mask1 * dest, axis=1, keepdims=True).astype(jnp.int32)
    p1_ref[...] = jnp.sum(mask2 * dest, axis=1, keepdims=True).astype(jnp.int32)
    cnt_ref[...] = counts.astype(jnp.int32)


def _dispatch_body(x_hbm, p0_hbm, p1_hbm, xs_hbm, pos_v, xbuf_v, sem):
    wid = lax.axis_index("s") * NC + lax.axis_index("c")
    base = wid * 64
    pltpu.sync_copy(x_hbm.at[pl.ds(base, 64)], xbuf_v)
    pltpu.sync_copy(p0_hbm.at[pl.ds(base, 64)], pos_v)
    pltpu.async_copy(xbuf_v, xs_hbm.at[pos_v], sem).wait()
    pltpu.sync_copy(p1_hbm.at[pl.ds(base, 64)], pos_v)
    pltpu.async_copy(xbuf_v, xs_hbm.at[pos_v], sem).wait()


def _ffn_group_body(gid_ref, xs_ref, w1_ref, b1_ref, w2_ref, b2_ref, y_ref):
    h = jnp.dot(xs_ref[...], w1_ref[0], preferred_element_type=jnp.float32)
    h = jax.nn.gelu(h + b1_ref[0, 0, :][None, :])
    y = jnp.dot(h, w2_ref[0], preferred_element_type=jnp.float32)
    y_ref[...] = y + b2_ref[0, 0, :][None, :]


def _gather_back_body(p0_hbm, p1_hbm, ys_hbm, y0_hbm, y1_hbm, pos_v, ybuf_v, sem):
    wid = lax.axis_index("s") * NC + lax.axis_index("c")
    base = wid * 64
    pltpu.sync_copy(p0_hbm.at[pl.ds(base, 64)], pos_v)
    pltpu.async_copy(ys_hbm.at[pos_v], ybuf_v, sem).wait()
    pltpu.sync_copy(ybuf_v, y0_hbm.at[pl.ds(base, 64)])
    pltpu.sync_copy(p1_hbm.at[pl.ds(base, 64)], pos_v)
    pltpu.async_copy(ys_hbm.at[pos_v], ybuf_v, sem).wait()
    pltpu.sync_copy(ybuf_v, y1_hbm.at[pl.ds(base, 64)])


def _combine_body(w0_ref, w1_ref, y0_ref, y1_ref, out_ref):
    out_ref[...] = w0_ref[...] * y0_ref[...] + w1_ref[...] * y1_ref[...]


def kernel(inputs, Wg, W1, b1, W2, b2):
    B, S, D = inputs.shape
    T = B * S
    F = W1.shape[2]
    x = inputs.reshape(T, D)
    NB = T * K // BLK + E
    PMAX = NB * BLK

    p0c, p1c, w0c, w1c, counts = pl.pallas_call(
        _gate_body,
        out_shape=(
            jax.ShapeDtypeStruct((T, 1), jnp.int32),
            jax.ShapeDtypeStruct((T, 1), jnp.int32),
            jax.ShapeDtypeStruct((T, 1), jnp.float32),
            jax.ShapeDtypeStruct((T, 1), jnp.float32),
            jax.ShapeDtypeStruct((1, E), jnp.int32),
        ),
    )(x, Wg)
    p0 = p0c.reshape(T)
    p1 = p1c.reshape(T)

    cnt = counts.reshape(E)
    nblk = (cnt + BLK - 1) // BLK
    ends = jnp.cumsum(nblk)
    bid = jnp.arange(NB, dtype=jnp.int32)
    gid = jnp.sum((bid[:, None] >= ends[None, :]).astype(jnp.int32), axis=1)
    last_ne = jnp.max(jnp.where(cnt > 0, jnp.arange(E, dtype=jnp.int32), 0))
    gid = jnp.minimum(gid, last_ne)

    mesh = plsc.VectorSubcoreMesh(core_axis_name="c", subcore_axis_name="s")
    x_sorted = pl.kernel(
        _dispatch_body,
        mesh=mesh,
        out_type=jax.ShapeDtypeStruct((PMAX, D), jnp.float32),
        scratch_types=[
            pltpu.VMEM((64,), jnp.int32),
            pltpu.VMEM((64, D), jnp.float32),
            pltpu.SemaphoreType.DMA,
        ],
    )(x, p0, p1)

    y_sorted = pl.pallas_call(
        _ffn_group_body,
        grid_spec=pltpu.PrefetchScalarGridSpec(
            num_scalar_prefetch=1,
            grid=(NB,),
            in_specs=[
                pl.BlockSpec((BLK, D), lambda i, g: (i, 0)),
                pl.BlockSpec((1, D, F), lambda i, g: (g[i], 0, 0)),
                pl.BlockSpec((1, 1, F), lambda i, g: (g[i], 0, 0)),
                pl.BlockSpec((1, F, D), lambda i, g: (g[i], 0, 0)),
                pl.BlockSpec((1, 1, D), lambda i, g: (g[i], 0, 0)),
            ],
            out_specs=pl.BlockSpec((BLK, D), lambda i, g: (i, 0)),
        ),
        out_shape=jax.ShapeDtypeStruct((PMAX, D), jnp.float32),
    )(gid, x_sorted, W1, b1.reshape(E, 1, F), W2, b2.reshape(E, 1, D))

    y0, y1 = pl.kernel(
        _gather_back_body,
        mesh=mesh,
        out_type=(
            jax.ShapeDtypeStruct((T, D), jnp.float32),
            jax.ShapeDtypeStruct((T, D), jnp.float32),
        ),
        scratch_types=[
            pltpu.VMEM((64,), jnp.int32),
            pltpu.VMEM((64, D), jnp.float32),
            pltpu.SemaphoreType.DMA,
        ],
    )(p0, p1, y_sorted)

    TB = 512
    out = pl.pallas_call(
        _combine_body,
        grid=(T // TB,),
        in_specs=[
            pl.BlockSpec((TB, 1), lambda i: (i, 0)),
            pl.BlockSpec((TB, 1), lambda i: (i, 0)),
            pl.BlockSpec((TB, D), lambda i: (i, 0)),
            pl.BlockSpec((TB, D), lambda i: (i, 0)),
        ],
        out_specs=pl.BlockSpec((TB, D), lambda i: (i, 0)),
        out_shape=jax.ShapeDtypeStruct((T, D), jnp.float32),
    )(w0c, w1c, y0, y1)

    return out.reshape(B, S, D)
